# baseline (speedup 1.0000x reference)
import jax
import jax.numpy as jnp
from jax import lax
from jax.experimental import pallas as pl
from jax.experimental.pallas import tpu as pltpu, tpu_sc as plsc

_B = 16384
_DIM = 32
_INFO = plsc.get_sparse_core_info()
_NC = _INFO.num_cores
_NS = _INFO.num_subcores
_NW = _NC * _NS
_BPW = _B // _NW
_CHUNK = 128
_NCHUNK = _BPW // _CHUNK


def _mf_body(user_ref, item_ref, uet_ref, iet_ref, ub_ref, ib_ref, gb_ref,
             out_ref,
             idx_u, idx_i, cu, ci, bu, bi, gb_v, out_v, sem):
    wid = lax.axis_index("s") * _NC + lax.axis_index("c")
    base = wid * _BPW

    pltpu.sync_copy(user_ref.at[wid], idx_u)
    pltpu.sync_copy(item_ref.at[wid], idx_i)
    pltpu.sync_copy(gb_ref, gb_v)

    bias_copies = []
    for j in range(_NCHUNK):
        bias_copies.append(pltpu.async_copy(ub_ref.at[idx_u.at[j]], bu.at[j], sem))
        bias_copies.append(pltpu.async_copy(ib_ref.at[idx_i.at[j]], bi.at[j], sem))

    pending = []
    for d in range(_DIM):
        fired = []
        for j in range(_NCHUNK):
            fired.append(pltpu.async_copy(uet_ref.at[d].at[idx_u.at[j]],
                                          cu.at[d].at[j], sem))
            fired.append(pltpu.async_copy(iet_ref.at[d].at[idx_i.at[j]],
                                          ci.at[d].at[j], sem))
        for c in pending:
            c.wait()
        pending = fired
    for c in pending:
        c.wait()
    for c in bias_copies:
        c.wait()

    gb = gb_v[...]

    for j in range(_NCHUNK):
        def body(g, carry, j=j):
            sl = pl.ds(g * 16, 16)
            acc = bu[j, sl] + bi[j, sl] + gb
            for d in range(_DIM):
                acc = acc + cu[d, j, sl] * ci[d, j, sl]
            out_v[pl.ds(j * _CHUNK + g * 16, 16)] = acc
            return carry
        lax.fori_loop(0, _CHUNK // 16, body, 0)

    pltpu.sync_copy(out_v, out_ref.at[pl.ds(base, _BPW)])


def kernel(user, item, user_emb, item_emb, user_bias, item_bias, global_bias):
    user = user.astype(jnp.int32).reshape(_NW, _NCHUNK, _CHUNK)
    item = item.astype(jnp.int32).reshape(_NW, _NCHUNK, _CHUNK)
    uet = user_emb.T
    iet = item_emb.T
    user_bias = user_bias.reshape(-1)
    item_bias = item_bias.reshape(-1)
    gb = jnp.broadcast_to(global_bias.astype(jnp.float32), (16,))

    mesh = plsc.VectorSubcoreMesh(core_axis_name="c", subcore_axis_name="s")
    f = pl.kernel(
        _mf_body,
        out_type=jax.ShapeDtypeStruct((_B,), jnp.float32),
        mesh=mesh,
        compiler_params=pltpu.CompilerParams(needs_layout_passes=False,
                                             use_tc_tiling_on_sc=False),
        scratch_types=[
            pltpu.VMEM((_NCHUNK, _CHUNK), jnp.int32),
            pltpu.VMEM((_NCHUNK, _CHUNK), jnp.int32),
            pltpu.VMEM((_DIM, _NCHUNK, _CHUNK), jnp.float32),
            pltpu.VMEM((_DIM, _NCHUNK, _CHUNK), jnp.float32),
            pltpu.VMEM((_NCHUNK, _CHUNK), jnp.float32),
            pltpu.VMEM((_NCHUNK, _CHUNK), jnp.float32),
            pltpu.VMEM((16,), jnp.float32),
            pltpu.VMEM((_BPW,), jnp.float32),
            pltpu.SemaphoreType.DMA,
        ],
    )
    return f(user, item, uet, iet, user_bias, item_bias, gb)


# speedup vs baseline: 3.1719x; 3.1719x over previous
import jax
import jax.numpy as jnp
from jax import lax
from jax.experimental import pallas as pl
from jax.experimental.pallas import tpu as pltpu, tpu_sc as plsc

_B = 16384
_DIM = 32
_N = 1000000
_INFO = plsc.get_sparse_core_info()
_NC = _INFO.num_cores
_NS = _INFO.num_subcores
_NW = _NC * _NS
_BPW = _B // _NW
_CHUNK = 128
_NCHUNK = _BPW // _CHUNK
_TW = 2048


def _tr_body(in_ref, out_ref):
    out_ref[...] = in_ref[...].T


def _transpose_table(et):
    grid = (_N + _TW - 1) // _TW
    return pl.pallas_call(
        _tr_body,
        grid=(grid,),
        in_specs=[pl.BlockSpec((_DIM, _TW), lambda i: (0, i))],
        out_specs=pl.BlockSpec((_TW, _DIM), lambda i: (i, 0)),
        out_shape=jax.ShapeDtypeStruct((_N, _DIM), jnp.float32),
    )(et)


def _mf_body(user_ref, item_ref, ue_ref, ie_ref, ub_ref, ib_ref, gb_ref,
             out_ref,
             idx_u, idx_i, rows_u, rows_i, bu, bi, gb_v, out_v, sem):
    wid = lax.axis_index("s") * _NC + lax.axis_index("c")
    base = wid * _BPW

    pltpu.sync_copy(user_ref.at[wid], idx_u)
    pltpu.sync_copy(item_ref.at[wid], idx_i)
    pltpu.sync_copy(gb_ref, gb_v)

    copies = []
    for j in range(_NCHUNK):
        copies.append(pltpu.async_copy(ue_ref.at[idx_u.at[j]], rows_u.at[j], sem))
        copies.append(pltpu.async_copy(ie_ref.at[idx_i.at[j]], rows_i.at[j], sem))
        copies.append(pltpu.async_copy(ub_ref.at[idx_u.at[j]], bu.at[j], sem))
        copies.append(pltpu.async_copy(ib_ref.at[idx_i.at[j]], bi.at[j], sem))
    for c in copies:
        c.wait()

    gb = gb_v[...]
    lane = lax.iota(jnp.int32, 16)

    for j in range(_NCHUNK):
        def body(g, carry, j=j):
            acc = jnp.zeros((16,), jnp.float32)
            for k in range(16):
                r = g * 16 + k
                ua = rows_u[j, r, pl.ds(0, 16)]
                ub2 = rows_u[j, r, pl.ds(16, 16)]
                ia = rows_i[j, r, pl.ds(0, 16)]
                ib2 = rows_i[j, r, pl.ds(16, 16)]
                t = ua * ia + ub2 * ib2
                s = jnp.sum(t)
                acc = jnp.where(lane == k, s, acc)
            bvec = bu[j, pl.ds(g * 16, 16)] + bi[j, pl.ds(g * 16, 16)]
            out_v[pl.ds(j * _CHUNK + g * 16, 16)] = acc + bvec + gb
            return carry
        lax.fori_loop(0, _CHUNK // 16, body, 0)

    pltpu.sync_copy(out_v, out_ref.at[pl.ds(base, _BPW)])


def kernel(user, item, user_emb, item_emb, user_bias, item_bias, global_bias):
    user = user.astype(jnp.int32).reshape(_NW, _NCHUNK, _CHUNK)
    item = item.astype(jnp.int32).reshape(_NW, _NCHUNK, _CHUNK)
    ue_rm = _transpose_table(user_emb.T)
    ie_rm = _transpose_table(item_emb.T)
    user_bias = user_bias.reshape(-1)
    item_bias = item_bias.reshape(-1)
    gb = jnp.broadcast_to(global_bias.astype(jnp.float32), (16,))

    mesh = plsc.VectorSubcoreMesh(core_axis_name="c", subcore_axis_name="s")
    f = pl.kernel(
        _mf_body,
        out_type=jax.ShapeDtypeStruct((_B,), jnp.float32),
        mesh=mesh,
        compiler_params=pltpu.CompilerParams(needs_layout_passes=False,
                                             use_tc_tiling_on_sc=False),
        scratch_types=[
            pltpu.VMEM((_NCHUNK, _CHUNK), jnp.int32),
            pltpu.VMEM((_NCHUNK, _CHUNK), jnp.int32),
            pltpu.VMEM((_NCHUNK, _CHUNK, _DIM), jnp.float32),
            pltpu.VMEM((_NCHUNK, _CHUNK, _DIM), jnp.float32),
            pltpu.VMEM((_NCHUNK, _CHUNK), jnp.float32),
            pltpu.VMEM((_NCHUNK, _CHUNK), jnp.float32),
            pltpu.VMEM((16,), jnp.float32),
            pltpu.VMEM((_BPW,), jnp.float32),
            pltpu.SemaphoreType.DMA,
        ],
    )
    return f(user, item, ue_rm, ie_rm, user_bias, item_bias, gb)


# speedup vs baseline: 4.3982x; 1.3866x over previous
import jax
import jax.numpy as jnp
from jax import lax
from jax.experimental import pallas as pl
from jax.experimental.pallas import tpu as pltpu, tpu_sc as plsc

_B = 16384
_DIM = 32
_N = 1000000
_INFO = plsc.get_sparse_core_info()
_NC = _INFO.num_cores
_NS = _INFO.num_subcores
_NW = _NC * _NS
_BPW = _B // _NW
_CHUNK = 128
_NCHUNK = _BPW // _CHUNK
_TW = 8192


def _tr_body(ue_ref, ie_ref, uo_ref, io_ref):
    eye = jnp.eye(_DIM, dtype=jnp.float32)
    dims = (((0,), (0,)), ((), ()))
    uo_ref[...] = lax.dot_general(ue_ref[...], eye, dims,
                                  preferred_element_type=jnp.float32)
    io_ref[...] = lax.dot_general(ie_ref[...], eye, dims,
                                  preferred_element_type=jnp.float32)


def _transpose_tables(uet, iet):
    grid = (_N + _TW - 1) // _TW
    return pl.pallas_call(
        _tr_body,
        grid=(grid,),
        in_specs=[pl.BlockSpec((_DIM, _TW), lambda i: (0, i)),
                  pl.BlockSpec((_DIM, _TW), lambda i: (0, i))],
        out_specs=[pl.BlockSpec((_TW, _DIM), lambda i: (i, 0)),
                   pl.BlockSpec((_TW, _DIM), lambda i: (i, 0))],
        out_shape=[jax.ShapeDtypeStruct((_N, _DIM), jnp.float32),
                   jax.ShapeDtypeStruct((_N, _DIM), jnp.float32)],
    )(uet, iet)


def _mf_body(user_ref, item_ref, ue_ref, ie_ref, ub_ref, ib_ref, gb_ref,
             out_ref,
             idx_u, idx_i, rows_u, rows_i, bu, bi, gb_v, out_v, sem):
    wid = lax.axis_index("s") * _NC + lax.axis_index("c")
    base = wid * _BPW

    pltpu.sync_copy(user_ref.at[wid], idx_u)
    pltpu.sync_copy(item_ref.at[wid], idx_i)
    pltpu.sync_copy(gb_ref, gb_v)

    copies = []
    for j in range(_NCHUNK):
        copies.append(pltpu.async_copy(ue_ref.at[idx_u.at[j]], rows_u.at[j], sem))
        copies.append(pltpu.async_copy(ie_ref.at[idx_i.at[j]], rows_i.at[j], sem))
        copies.append(pltpu.async_copy(ub_ref.at[idx_u.at[j]], bu.at[j], sem))
        copies.append(pltpu.async_copy(ib_ref.at[idx_i.at[j]], bi.at[j], sem))
    for c in copies:
        c.wait()

    gb = gb_v[...]
    lane = lax.iota(jnp.int32, 16)

    for j in range(_NCHUNK):
        def body(g, carry, j=j):
            acc = jnp.zeros((16,), jnp.float32)
            for k in range(16):
                r = g * 16 + k
                ua = rows_u[j, r, pl.ds(0, 16)]
                ub2 = rows_u[j, r, pl.ds(16, 16)]
                ia = rows_i[j, r, pl.ds(0, 16)]
                ib2 = rows_i[j, r, pl.ds(16, 16)]
                t = ua * ia + ub2 * ib2
                s = jnp.sum(t)
                acc = jnp.where(lane == k, s, acc)
            bvec = bu[j, pl.ds(g * 16, 16)] + bi[j, pl.ds(g * 16, 16)]
            out_v[pl.ds(j * _CHUNK + g * 16, 16)] = acc + bvec + gb
            return carry
        lax.fori_loop(0, _CHUNK // 16, body, 0)

    pltpu.sync_copy(out_v, out_ref.at[pl.ds(base, _BPW)])


def kernel(user, item, user_emb, item_emb, user_bias, item_bias, global_bias):
    user = user.astype(jnp.int32).reshape(_NW, _NCHUNK, _CHUNK)
    item = item.astype(jnp.int32).reshape(_NW, _NCHUNK, _CHUNK)
    ue_rm, ie_rm = _transpose_tables(user_emb.T, item_emb.T)
    user_bias = user_bias.reshape(-1)
    item_bias = item_bias.reshape(-1)
    gb = jnp.broadcast_to(global_bias.astype(jnp.float32), (16,))

    mesh = plsc.VectorSubcoreMesh(core_axis_name="c", subcore_axis_name="s")
    f = pl.kernel(
        _mf_body,
        out_type=jax.ShapeDtypeStruct((_B,), jnp.float32),
        mesh=mesh,
        compiler_params=pltpu.CompilerParams(needs_layout_passes=False,
                                             use_tc_tiling_on_sc=False),
        scratch_types=[
            pltpu.VMEM((_NCHUNK, _CHUNK), jnp.int32),
            pltpu.VMEM((_NCHUNK, _CHUNK), jnp.int32),
            pltpu.VMEM((_NCHUNK, _CHUNK, _DIM), jnp.float32),
            pltpu.VMEM((_NCHUNK, _CHUNK, _DIM), jnp.float32),
            pltpu.VMEM((_NCHUNK, _CHUNK), jnp.float32),
            pltpu.VMEM((_NCHUNK, _CHUNK), jnp.float32),
            pltpu.VMEM((16,), jnp.float32),
            pltpu.VMEM((_BPW,), jnp.float32),
            pltpu.SemaphoreType.DMA,
        ],
    )
    return f(user, item, ue_rm, ie_rm, user_bias, item_bias, gb)


# speedup vs baseline: 8.4278x; 1.9162x over previous
import jax
import jax.numpy as jnp
from jax import lax
from jax.experimental import pallas as pl
from jax.experimental.pallas import tpu as pltpu, tpu_sc as plsc

_B = 16384
_DIM = 32
_N = 1000000
_SEG = 262144
_NSEG = 4
_INFO = plsc.get_sparse_core_info()
_NC = _INFO.num_cores
_NS = _INFO.num_subcores
_NW = _NC * _NS
_BPW = _B // _NW
_CHUNK = 128
_NCHUNK = _BPW // _CHUNK
_TW = 4096
_NB = _SEG // _TW


def _repack_body(*refs):
    ins = refs[:2 * _NSEG]
    uo_ref, io_ref = refs[2 * _NSEG:]
    eye = jnp.eye(_DIM, dtype=jnp.float32)
    dims = (((0,), (0,)), ((), ()))
    for a in range(_NSEG):
        uo_ref[:, 32 * a:32 * (a + 1)] = lax.dot_general(
            ins[a][...], eye, dims, preferred_element_type=jnp.float32)
        io_ref[:, 32 * a:32 * (a + 1)] = lax.dot_general(
            ins[_NSEG + a][...], eye, dims, preferred_element_type=jnp.float32)


def _repack_tables(uet, iet):
    last = _N // _TW
    in_specs = []
    for _t in range(2):
        for a in range(_NSEG):
            in_specs.append(
                pl.BlockSpec((_DIM, _TW),
                             lambda i, a=a: (0, jnp.minimum(a * _NB + i, last))))
    return pl.pallas_call(
        _repack_body,
        grid=(_NB,),
        in_specs=in_specs,
        out_specs=[pl.BlockSpec((_TW, 128), lambda i: (i, 0)),
                   pl.BlockSpec((_TW, 128), lambda i: (i, 0))],
        out_shape=[jax.ShapeDtypeStruct((_SEG, 128), jnp.float32),
                   jax.ShapeDtypeStruct((_SEG, 128), jnp.float32)],
    )(*([uet] * _NSEG + [iet] * _NSEG))


def _mf_body(user_ref, item_ref, ue_ref, ie_ref, ub_ref, ib_ref, gb_ref,
             out_ref,
             idx_u, idx_i, qq_u, qq_i, ao_u, ao_i,
             rows_u, rows_i, bu, bi, gb_v, out_v, sem):
    wid = lax.axis_index("s") * _NC + lax.axis_index("c")
    base = wid * _BPW

    pltpu.sync_copy(user_ref.at[wid], idx_u)
    pltpu.sync_copy(item_ref.at[wid], idx_i)
    pltpu.sync_copy(gb_ref, gb_v)

    for j in range(_NCHUNK):
        for g in range(_CHUNK // 16):
            sl = pl.ds(g * 16, 16)
            u16 = idx_u[j, sl]
            qq_u[j, sl] = jnp.bitwise_and(u16, _SEG - 1)
            ao_u[j, sl] = jnp.right_shift(u16, 18) * 32
            i16 = idx_i[j, sl]
            qq_i[j, sl] = jnp.bitwise_and(i16, _SEG - 1)
            ao_i[j, sl] = jnp.right_shift(i16, 18) * 32

    bias_copies = []
    for j in range(_NCHUNK):
        bias_copies.append(pltpu.async_copy(ub_ref.at[idx_u.at[j]], bu.at[j], sem))
        bias_copies.append(pltpu.async_copy(ib_ref.at[idx_i.at[j]], bi.at[j], sem))

    def fire(j):
        b = j % 2
        return [pltpu.async_copy(ue_ref.at[qq_u.at[j]], rows_u.at[b], sem),
                pltpu.async_copy(ie_ref.at[qq_i.at[j]], rows_i.at[b], sem)]

    gb = gb_v[...]
    lane = lax.iota(jnp.int32, 16)
    pending = {0: fire(0), 1: fire(1)}

    for j in range(_NCHUNK):
        for c in pending.pop(j):
            c.wait()
        b = j % 2

        def body(g, carry, j=j, b=b):
            acc = jnp.zeros((16,), jnp.float32)
            aou16 = ao_u[j, pl.ds(g * 16, 16)]
            aoi16 = ao_i[j, pl.ds(g * 16, 16)]
            zero = lane * 0
            bvec = zero + b
            for k in range(16):
                r = g * 16 + k
                rvec = zero + r
                ou = aou16[k]
                oi = aoi16[k]
                ua = plsc.load_gather(rows_u, [bvec, rvec, lane + ou])
                ub2 = plsc.load_gather(rows_u, [bvec, rvec, lane + ou + 16])
                ia = plsc.load_gather(rows_i, [bvec, rvec, lane + oi])
                ib2 = plsc.load_gather(rows_i, [bvec, rvec, lane + oi + 16])
                t = ua * ia + ub2 * ib2
                s = jnp.sum(t)
                acc = jnp.where(lane == k, s, acc)
            bvec = bu[j, pl.ds(g * 16, 16)] + bi[j, pl.ds(g * 16, 16)]
            out_v[pl.ds(j * _CHUNK + g * 16, 16)] = acc + bvec + gb
            return carry

        lax.fori_loop(0, _CHUNK // 16, body, 0)
        if j + 2 < _NCHUNK:
            pending[j + 2] = fire(j + 2)

    for c in bias_copies:
        c.wait()
    pltpu.sync_copy(out_v, out_ref.at[pl.ds(base, _BPW)])


def kernel(user, item, user_emb, item_emb, user_bias, item_bias, global_bias):
    user = user.astype(jnp.int32).reshape(_NW, _NCHUNK, _CHUNK)
    item = item.astype(jnp.int32).reshape(_NW, _NCHUNK, _CHUNK)
    ue_c, ie_c = _repack_tables(user_emb.T, item_emb.T)
    user_bias = user_bias.reshape(-1)
    item_bias = item_bias.reshape(-1)
    gb = jnp.broadcast_to(global_bias.astype(jnp.float32), (16,))

    mesh = plsc.VectorSubcoreMesh(core_axis_name="c", subcore_axis_name="s")
    f = pl.kernel(
        _mf_body,
        out_type=jax.ShapeDtypeStruct((_B,), jnp.float32),
        mesh=mesh,
        compiler_params=pltpu.CompilerParams(needs_layout_passes=False,
                                             use_tc_tiling_on_sc=False),
        scratch_types=[
            pltpu.VMEM((_NCHUNK, _CHUNK), jnp.int32),
            pltpu.VMEM((_NCHUNK, _CHUNK), jnp.int32),
            pltpu.VMEM((_NCHUNK, _CHUNK), jnp.int32),
            pltpu.VMEM((_NCHUNK, _CHUNK), jnp.int32),
            pltpu.VMEM((_NCHUNK, _CHUNK), jnp.int32),
            pltpu.VMEM((_NCHUNK, _CHUNK), jnp.int32),
            pltpu.VMEM((2, _CHUNK, 128), jnp.float32),
            pltpu.VMEM((2, _CHUNK, 128), jnp.float32),
            pltpu.VMEM((_NCHUNK, _CHUNK), jnp.float32),
            pltpu.VMEM((_NCHUNK, _CHUNK), jnp.float32),
            pltpu.VMEM((16,), jnp.float32),
            pltpu.VMEM((_BPW,), jnp.float32),
            pltpu.SemaphoreType.DMA,
        ],
    )
    return f(user, item, ue_c, ie_c, user_bias, item_bias, gb)


# speedup vs baseline: 8.4820x; 1.0064x over previous
import jax
import jax.numpy as jnp
from jax import lax
from jax.experimental import pallas as pl
from jax.experimental.pallas import tpu as pltpu, tpu_sc as plsc

_B = 16384
_DIM = 32
_N = 1000000
_SEG = 262144
_NSEG = 4
_INFO = plsc.get_sparse_core_info()
_NC = _INFO.num_cores
_NS = _INFO.num_subcores
_NW = _NC * _NS
_BPW = _B // _NW
_CHUNK = 128
_NCHUNK = _BPW // _CHUNK
_TW = 8192
_NB = _SEG // _TW


def _repack_body(*refs):
    ins = refs[:2 * _NSEG]
    uo_ref, io_ref = refs[2 * _NSEG:]
    eye = jnp.eye(_DIM, dtype=jnp.float32)
    dims = (((0,), (0,)), ((), ()))
    for a in range(_NSEG):
        uo_ref[:, 32 * a:32 * (a + 1)] = lax.dot_general(
            ins[a][...], eye, dims, preferred_element_type=jnp.float32)
        io_ref[:, 32 * a:32 * (a + 1)] = lax.dot_general(
            ins[_NSEG + a][...], eye, dims, preferred_element_type=jnp.float32)


def _repack_tables(uet, iet):
    last = _N // _TW
    in_specs = []
    for _t in range(2):
        for a in range(_NSEG):
            in_specs.append(
                pl.BlockSpec((_DIM, _TW),
                             lambda i, a=a: (0, jnp.minimum(a * _NB + i, last))))
    return pl.pallas_call(
        _repack_body,
        grid=(_NB,),
        in_specs=in_specs,
        out_specs=[pl.BlockSpec((_TW, 128), lambda i: (i, 0)),
                   pl.BlockSpec((_TW, 128), lambda i: (i, 0))],
        out_shape=[jax.ShapeDtypeStruct((_SEG, 128), jnp.float32),
                   jax.ShapeDtypeStruct((_SEG, 128), jnp.float32)],
    )(*([uet] * _NSEG + [iet] * _NSEG))


def _mf_body(user_ref, item_ref, ue_ref, ie_ref, ub_ref, ib_ref, gb_ref,
             out_ref,
             idx_u, idx_i, qq_u, qq_i, ao_u, ao_i,
             rows_u, rows_i, bu, bi, gb_v, out_v, sem):
    wid = lax.axis_index("s") * _NC + lax.axis_index("c")
    base = wid * _BPW

    pltpu.sync_copy(user_ref.at[wid], idx_u)
    pltpu.sync_copy(item_ref.at[wid], idx_i)
    pltpu.sync_copy(gb_ref, gb_v)

    for j in range(_NCHUNK):
        for g in range(_CHUNK // 16):
            sl = pl.ds(g * 16, 16)
            u16 = idx_u[j, sl]
            qq_u[j, sl] = jnp.bitwise_and(u16, _SEG - 1)
            ao_u[j, sl] = jnp.right_shift(u16, 18) * 32
            i16 = idx_i[j, sl]
            qq_i[j, sl] = jnp.bitwise_and(i16, _SEG - 1)
            ao_i[j, sl] = jnp.right_shift(i16, 18) * 32

    bias_copies = []
    for j in range(_NCHUNK):
        bias_copies.append(pltpu.async_copy(ub_ref.at[idx_u.at[j]], bu.at[j], sem))
        bias_copies.append(pltpu.async_copy(ib_ref.at[idx_i.at[j]], bi.at[j], sem))

    def fire(j):
        b = j % 2
        return [pltpu.async_copy(ue_ref.at[qq_u.at[j]], rows_u.at[b], sem),
                pltpu.async_copy(ie_ref.at[qq_i.at[j]], rows_i.at[b], sem)]

    gb = gb_v[...]
    lane = lax.iota(jnp.int32, 16)
    pending = {0: fire(0), 1: fire(1)}

    for j in range(_NCHUNK):
        for c in pending.pop(j):
            c.wait()
        b = j % 2

        def body(g, carry, j=j, b=b):
            acc = jnp.zeros((16,), jnp.float32)
            aou16 = ao_u[j, pl.ds(g * 16, 16)]
            aoi16 = ao_i[j, pl.ds(g * 16, 16)]
            zero = lane * 0
            bvec = zero + b
            for k in range(16):
                r = g * 16 + k
                rvec = zero + r
                ou = aou16[k]
                oi = aoi16[k]
                ua = plsc.load_gather(rows_u, [bvec, rvec, lane + ou])
                ub2 = plsc.load_gather(rows_u, [bvec, rvec, lane + ou + 16])
                ia = plsc.load_gather(rows_i, [bvec, rvec, lane + oi])
                ib2 = plsc.load_gather(rows_i, [bvec, rvec, lane + oi + 16])
                t = ua * ia + ub2 * ib2
                s = jnp.sum(t)
                acc = jnp.where(lane == k, s, acc)
            bvec = bu[j, pl.ds(g * 16, 16)] + bi[j, pl.ds(g * 16, 16)]
            out_v[pl.ds(j * _CHUNK + g * 16, 16)] = acc + bvec + gb
            return carry

        lax.fori_loop(0, _CHUNK // 16, body, 0)
        if j + 2 < _NCHUNK:
            pending[j + 2] = fire(j + 2)

    for c in bias_copies:
        c.wait()
    pltpu.sync_copy(out_v, out_ref.at[pl.ds(base, _BPW)])


def kernel(user, item, user_emb, item_emb, user_bias, item_bias, global_bias):
    user = user.astype(jnp.int32).reshape(_NW, _NCHUNK, _CHUNK)
    item = item.astype(jnp.int32).reshape(_NW, _NCHUNK, _CHUNK)
    ue_c, ie_c = _repack_tables(user_emb.T, item_emb.T)
    user_bias = user_bias.reshape(-1)
    item_bias = item_bias.reshape(-1)
    gb = jnp.broadcast_to(global_bias.astype(jnp.float32), (16,))

    mesh = plsc.VectorSubcoreMesh(core_axis_name="c", subcore_axis_name="s")
    f = pl.kernel(
        _mf_body,
        out_type=jax.ShapeDtypeStruct((_B,), jnp.float32),
        mesh=mesh,
        compiler_params=pltpu.CompilerParams(needs_layout_passes=False,
                                             use_tc_tiling_on_sc=False),
        scratch_types=[
            pltpu.VMEM((_NCHUNK, _CHUNK), jnp.int32),
            pltpu.VMEM((_NCHUNK, _CHUNK), jnp.int32),
            pltpu.VMEM((_NCHUNK, _CHUNK), jnp.int32),
            pltpu.VMEM((_NCHUNK, _CHUNK), jnp.int32),
            pltpu.VMEM((_NCHUNK, _CHUNK), jnp.int32),
            pltpu.VMEM((_NCHUNK, _CHUNK), jnp.int32),
            pltpu.VMEM((2, _CHUNK, 128), jnp.float32),
            pltpu.VMEM((2, _CHUNK, 128), jnp.float32),
            pltpu.VMEM((_NCHUNK, _CHUNK), jnp.float32),
            pltpu.VMEM((_NCHUNK, _CHUNK), jnp.float32),
            pltpu.VMEM((16,), jnp.float32),
            pltpu.VMEM((_BPW,), jnp.float32),
            pltpu.SemaphoreType.DMA,
        ],
    )
    return f(user, item, ue_c, ie_c, user_bias, item_bias, gb)
